# baseline (speedup 1.0000x reference)
import jax
import jax.numpy as jnp
from jax import lax
from jax.experimental import pallas as pl
from jax.experimental.pallas import tpu as pltpu
from jax.experimental.pallas import tpu_sc as plsc

N = 10000
E = 160000
DN = 128
DE = 16
U = 64

NC, NS, L = 2, 16, 16
NW = NC * NS
N_PAD = 10240
CHUNK = E // NW
CAP = 5008
NG = 10016
NFULL = CHUNK // L
TAIL = CHUNK - NFULL * L
EB = 16000

_mesh = plsc.VectorSubcoreMesh(core_axis_name="c", subcore_axis_name="s")
_HI = lax.Precision.HIGHEST
_sc_params = pltpu.CompilerParams(needs_layout_passes=False,
                                  use_tc_tiling_on_sc=False)


def _proj_body(us_ref, vs_ref, es2_ref, ku_ref, kv_ref, ke_ref, watt_ref,
               au_ref, av1_ref, av2_ref, ae_ref):
    i = pl.program_id(0)
    watt = watt_ref[...]
    we = jnp.dot(ke_ref[...], watt[192:256], precision=_HI)[:, 0]
    wrep = jnp.concatenate([we] * 8)
    r = lax.broadcasted_iota(jnp.int32, (DN, 8), 0)
    c = lax.broadcasted_iota(jnp.int32, (DN, 8), 1)
    wmat = jnp.where((r // DE) == c, wrep[:, None], 0.0)
    ae_ref[...] = jnp.dot(es2_ref[...], wmat, precision=_HI)

    @pl.when(i == 0)
    def _node_scalars():
        wu = jnp.dot(ku_ref[...], watt[0:64], precision=_HI)[:, 0]
        wv1 = jnp.dot(kv_ref[...], watt[64:128], precision=_HI)[:, 0]
        wv2 = jnp.dot(kv_ref[...], watt[128:192], precision=_HI)[:, 0]
        us = us_ref[...]
        vs = vs_ref[...]
        au_ref[...] = jnp.sum(us * wu[None, :], axis=1)
        av1_ref[...] = jnp.sum(vs * wv1[None, :], axis=1)
        av2_ref[...] = jnp.sum(vs * wv2[None, :], axis=1)


def _proj(us, vs, es2, ku, kv, ke, watt):
    g = 10
    eb = E // 8 // g
    return pl.pallas_call(
        _proj_body,
        grid=(g,),
        in_specs=[
            pl.BlockSpec((N, DN), lambda i: (0, 0)),
            pl.BlockSpec((N, DN), lambda i: (0, 0)),
            pl.BlockSpec((eb, DN), lambda i: (i, 0)),
            pl.BlockSpec((DN, U), lambda i: (0, 0)),
            pl.BlockSpec((DN, U), lambda i: (0, 0)),
            pl.BlockSpec((DE, U), lambda i: (0, 0)),
            pl.BlockSpec((4 * U, 1), lambda i: (0, 0)),
        ],
        out_specs=(
            pl.BlockSpec((N,), lambda i: (0,)),
            pl.BlockSpec((N,), lambda i: (0,)),
            pl.BlockSpec((N,), lambda i: (0,)),
            pl.BlockSpec((eb, 8), lambda i: (i, 0)),
        ),
        out_shape=(
            jax.ShapeDtypeStruct((N,), jnp.float32),
            jax.ShapeDtypeStruct((N,), jnp.float32),
            jax.ShapeDtypeStruct((N,), jnp.float32),
            jax.ShapeDtypeStruct((E // 8, 8), jnp.float32),
        ),
    )(us, vs, es2, ku, kv, ke, watt)


def _scores_body(e0_hbm, e1_hbm, e2_hbm, au_hbm, av1_hbm, av2_hbm, ae_hbm,
                 s_hbm, pcnt_hbm, psum_hbm,
                 e0v, e1v, e2v, aev, auv, av1v, av2v, sv, onesv, zbuf,
                 sh_cnt, sh_sum):
    cid = lax.axis_index("c")
    sid = lax.axis_index("s")
    wid = cid * NS + sid
    base = wid * CHUNK

    pltpu.sync_copy(e0_hbm.at[pl.ds(base, CHUNK)], e0v.at[pl.ds(0, CHUNK)])
    pltpu.sync_copy(e1_hbm.at[pl.ds(base, CHUNK)], e1v.at[pl.ds(0, CHUNK)])
    pltpu.sync_copy(e2_hbm.at[pl.ds(base, CHUNK)], e2v.at[pl.ds(0, CHUNK)])
    pltpu.sync_copy(ae_hbm.at[pl.ds(base, CHUNK)], aev.at[pl.ds(0, CHUNK)])
    pltpu.sync_copy(au_hbm, auv.at[pl.ds(0, N)])
    pltpu.sync_copy(av1_hbm, av1v.at[pl.ds(0, N)])
    pltpu.sync_copy(av2_hbm, av2v.at[pl.ds(0, N)])

    zero16 = jnp.zeros((L,), jnp.float32)
    one16 = jnp.ones((L,), jnp.float32)
    lane = lax.iota(jnp.int32, L)
    mtail = lane < TAIL

    def _fill(i, _):
        zbuf[pl.ds(i * L, L)] = zero16
        return 0
    lax.fori_loop(0, 40, _fill, 0)

    def _fill_ones(i, _):
        onesv[pl.ds(i * L, L)] = one16
        return 0
    lax.fori_loop(0, NFULL, _fill_ones, 0)
    onesv[pl.ds(NFULL * L, L)] = jnp.where(mtail, 1.0, 0.0)

    pltpu.sync_copy(zbuf, sh_cnt.at[pl.ds(sid * 640, 640)])
    pltpu.sync_copy(zbuf, sh_sum.at[pl.ds(sid * 640, 640)])

    def _score_vec(i0, i1, i2, ea):
        a = (plsc.load_gather(auv, [i0]) + plsc.load_gather(av1v, [i1])
             + plsc.load_gather(av2v, [i2]) + ea)
        a = jnp.where(a >= 0.0, a, 0.2 * a)
        return jnp.exp(jnp.clip(a, -2.0, 2.0))

    def _score(j, _):
        off = j * L
        sv[pl.ds(off, L)] = _score_vec(
            e0v[pl.ds(off, L)], e1v[pl.ds(off, L)], e2v[pl.ds(off, L)],
            aev[pl.ds(off, L)])
        return 0
    lax.fori_loop(0, NFULL, _score, 0)

    off = NFULL * L
    t0 = jnp.where(mtail, e0v[pl.ds(off, L)], 0)
    t1 = jnp.where(mtail, e1v[pl.ds(off, L)], 0)
    t2 = jnp.where(mtail, e2v[pl.ds(off, L)], 0)
    sv[pl.ds(off, L)] = _score_vec(t0, t1, t2, aev[pl.ds(off, L)])
    e0v[pl.ds(off, L)] = jnp.where(mtail, t0, N)

    pltpu.sync_copy(sv.at[pl.ds(0, CHUNK)], s_hbm.at[pl.ds(base, CHUNK)])

    plsc.subcore_barrier()
    pltpu.sync_copy(onesv, sh_cnt.at[e0v], add=True)
    pltpu.sync_copy(sv, sh_sum.at[e0v], add=True)
    plsc.subcore_barrier()

    pltpu.sync_copy(sh_cnt.at[pl.ds(sid * 640, 640)],
                    pcnt_hbm.at[pl.ds(cid * N_PAD + sid * 640, 640)])
    pltpu.sync_copy(sh_sum.at[pl.ds(sid * 640, 640)],
                    psum_hbm.at[pl.ds(cid * N_PAD + sid * 640, 640)])


def _scores(e0, e1, e2, au, av1, av2, ae):
    f = pl.kernel(
        _scores_body,
        out_type=(
            jax.ShapeDtypeStruct((E,), jnp.float32),
            jax.ShapeDtypeStruct((NC * N_PAD,), jnp.float32),
            jax.ShapeDtypeStruct((NC * N_PAD,), jnp.float32),
        ),
        mesh=_mesh,
        compiler_params=_sc_params,
        scratch_types=[
            pltpu.VMEM((CAP,), jnp.int32),
            pltpu.VMEM((CAP,), jnp.int32),
            pltpu.VMEM((CAP,), jnp.int32),
            pltpu.VMEM((CAP,), jnp.float32),
            pltpu.VMEM((N_PAD,), jnp.float32),
            pltpu.VMEM((N_PAD,), jnp.float32),
            pltpu.VMEM((N_PAD,), jnp.float32),
            pltpu.VMEM((CAP,), jnp.float32),
            pltpu.VMEM((CAP,), jnp.float32),
            pltpu.VMEM((640,), jnp.float32),
            pltpu.VMEM_SHARED((N_PAD,), jnp.float32),
            pltpu.VMEM_SHARED((N_PAD,), jnp.float32),
        ],
    )
    return f(e0, e1, e2, au, av1, av2, ae)


def _scatter_body(pcnt_hbm, psum_hbm, s_hbm, e0_hbm, es_hbm,
                  pacc_hbm,
                  ccv, ssumv, sv, e0v, rows, psv, zbuf,
                  sh_acc):
    cid = lax.axis_index("c")
    sid = lax.axis_index("s")
    wid = cid * NS + sid
    base = wid * CHUNK
    lane = lax.iota(jnp.int32, L)
    mtail = lane < TAIL

    pltpu.sync_copy(pcnt_hbm.at[pl.ds(0, NG)], ccv)
    pltpu.sync_copy(pcnt_hbm.at[pl.ds(N_PAD, NG)], ssumv)

    def _cc(j, carry):
        c = ccv[pl.ds(j * L, L)] + ssumv[pl.ds(j * L, L)]
        ccv[pl.ds(j * L, L)] = plsc.cumsum(c) + carry
        return carry + jnp.sum(c)
    lax.fori_loop(0, NG // L, _cc, jnp.float32(0.0))

    pltpu.sync_copy(psum_hbm.at[pl.ds(0, NG)], ssumv)
    pltpu.sync_copy(psum_hbm.at[pl.ds(N_PAD, NG)], psv)
    pltpu.sync_copy(s_hbm.at[pl.ds(base, CHUNK)], sv.at[pl.ds(0, CHUNK)])
    pltpu.sync_copy(e0_hbm.at[pl.ds(base, CHUNK)], e0v.at[pl.ds(0, CHUNK)])
    pltpu.sync_copy(es_hbm.at[pl.ds(base, CHUNK), :], rows.at[pl.ds(0, CHUNK), :])

    zero16 = jnp.zeros((L,), jnp.float32)

    def _fill(i, _):
        zbuf[i, :] = zero16
        return 0
    lax.fori_loop(0, 32, _fill, 0)

    for t in range(20):
        pltpu.sync_copy(zbuf, sh_acc.at[pl.ds(sid * 640 + t * 32, 32), :])

    def _search(j, _):
        off = j * L
        pos = (base + off + lane).astype(jnp.float32)
        lo = jnp.zeros((L,), jnp.int32)
        step = 8192
        while step >= 1:
            cand = lo + step
            idx = jnp.minimum(cand, N) - 1
            v = plsc.load_gather(ccv, [idx])
            take = (cand <= N) & (v <= pos)
            lo = jnp.where(take, cand, lo)
            step //= 2
        rep = plsc.load_gather(ssumv, [lo]) + plsc.load_gather(psv, [lo])
        sv[pl.ds(off, L)] = sv[pl.ds(off, L)] / rep
        return 0
    lax.fori_loop(0, NFULL + 1, _search, 0)

    off = NFULL * L
    e0v[pl.ds(off, L)] = jnp.where(mtail, e0v[pl.ds(off, L)], N)

    def _scale(j, _):
        wv = sv[pl.ds(j * L, L)]
        for r2 in range(L):
            r = j * L + r2
            rows[r, :] = rows[r, :] * jnp.full((L,), wv[r2])
        return 0
    lax.fori_loop(0, NFULL, _scale, 0)
    wtail = sv[pl.ds(off, L)]
    for r2 in range(TAIL):
        rows[off + r2, :] = rows[off + r2, :] * jnp.full((L,), wtail[r2])

    plsc.subcore_barrier()
    pltpu.sync_copy(rows, sh_acc.at[e0v], add=True)
    plsc.subcore_barrier()

    pltpu.sync_copy(sh_acc.at[pl.ds(sid * 640, 640), :],
                    pacc_hbm.at[pl.ds(cid * N_PAD + sid * 640, 640), :])


def _scatter(pcnt, psum, s, e0, es):
    f = pl.kernel(
        _scatter_body,
        out_type=jax.ShapeDtypeStruct((NC * N_PAD, DE), jnp.float32),
        mesh=_mesh,
        compiler_params=_sc_params,
        scratch_types=[
            pltpu.VMEM((NG,), jnp.float32),
            pltpu.VMEM((NG,), jnp.float32),
            pltpu.VMEM((CAP,), jnp.float32),
            pltpu.VMEM((CAP,), jnp.int32),
            pltpu.VMEM((CAP, DE), jnp.float32),
            pltpu.VMEM((NG,), jnp.float32),
            pltpu.VMEM((32, DE), jnp.float32),
            pltpu.VMEM_SHARED((N_PAD, DE), jnp.float32),
        ],
    )
    return f(pcnt, psum, s, e0, es)


def _out_body(pacc_ref, ke_ref, out_ref):
    acc = pacc_ref[0, :, :] + pacc_ref[1, :, :]
    res = jnp.dot(acc, ke_ref[...], precision=_HI)
    out_ref[...] = res[0:N, :]


def _out_mm(pacc, ke):
    return pl.pallas_call(
        _out_body,
        out_shape=jax.ShapeDtypeStruct((N, U), jnp.float32),
    )(pacc, ke)


def kernel(user_states, video_states, edge_states, hyper_edges,
           kernel_user, kernel_video, kernel_edge, kernel_user_attention):
    e0 = hyper_edges[:, 0]
    e1 = hyper_edges[:, 1]
    e2 = hyper_edges[:, 2]
    es2 = edge_states.reshape(E // 8, DN)
    esp = jnp.concatenate([edge_states, jnp.zeros((16, DE), jnp.float32)])

    au, av1, av2, ae2 = _proj(user_states, video_states, es2, kernel_user,
                              kernel_video, kernel_edge,
                              kernel_user_attention)
    s, pcnt, psum = _scores(e0, e1, e2, au, av1, av2, ae2.reshape(E))
    pacc = _scatter(pcnt, psum, s, e0, esp)
    return _out_mm(pacc.reshape(NC, N_PAD, DE), kernel_edge)


# speedup vs baseline: 1.3266x; 1.3266x over previous
import jax
import jax.numpy as jnp
from jax import lax
from jax.experimental import pallas as pl
from jax.experimental.pallas import tpu as pltpu
from jax.experimental.pallas import tpu_sc as plsc

N = 10000
E = 160000
DN = 128
DE = 16
U = 64

NC, NS, L = 2, 16, 16
NW = NC * NS
N_PAD = 10240
CHUNK = E // NW
CAP = 5008
NG = 10016
NFULL = CHUNK // L
TAIL = CHUNK - NFULL * L
EB = 16000

_mesh = plsc.VectorSubcoreMesh(core_axis_name="c", subcore_axis_name="s")
_HI = lax.Precision.HIGHEST
_sc_params = pltpu.CompilerParams(needs_layout_passes=False,
                                  use_tc_tiling_on_sc=False)


def _proj_body(us_ref, vs_ref, es2_ref, ku_ref, kv_ref, ke_ref, watt_ref,
               au_ref, av1_ref, av2_ref, ae_ref):
    i = pl.program_id(0)
    watt = watt_ref[...]
    we = jnp.dot(ke_ref[...], watt[192:256], precision=_HI)[:, 0]
    wrep = jnp.concatenate([we] * 8)
    r = lax.broadcasted_iota(jnp.int32, (DN, 8), 0)
    c = lax.broadcasted_iota(jnp.int32, (DN, 8), 1)
    wmat = jnp.where((r // DE) == c, wrep[:, None], 0.0)
    ae_ref[...] = jnp.dot(es2_ref[...], wmat, precision=_HI)

    @pl.when(i == 0)
    def _node_scalars():
        wu = jnp.dot(ku_ref[...], watt[0:64], precision=_HI)[:, 0]
        wv1 = jnp.dot(kv_ref[...], watt[64:128], precision=_HI)[:, 0]
        wv2 = jnp.dot(kv_ref[...], watt[128:192], precision=_HI)[:, 0]
        us = us_ref[...]
        vs = vs_ref[...]
        au_ref[...] = jnp.sum(us * wu[None, :], axis=1)
        av1_ref[...] = jnp.sum(vs * wv1[None, :], axis=1)
        av2_ref[...] = jnp.sum(vs * wv2[None, :], axis=1)


def _proj(us, vs, es2, ku, kv, ke, watt):
    g = 10
    eb = E // 8 // g
    return pl.pallas_call(
        _proj_body,
        grid=(g,),
        in_specs=[
            pl.BlockSpec((N, DN), lambda i: (0, 0)),
            pl.BlockSpec((N, DN), lambda i: (0, 0)),
            pl.BlockSpec((eb, DN), lambda i: (i, 0)),
            pl.BlockSpec((DN, U), lambda i: (0, 0)),
            pl.BlockSpec((DN, U), lambda i: (0, 0)),
            pl.BlockSpec((DE, U), lambda i: (0, 0)),
            pl.BlockSpec((4 * U, 1), lambda i: (0, 0)),
        ],
        out_specs=(
            pl.BlockSpec((N,), lambda i: (0,)),
            pl.BlockSpec((N,), lambda i: (0,)),
            pl.BlockSpec((N,), lambda i: (0,)),
            pl.BlockSpec((eb, 8), lambda i: (i, 0)),
        ),
        out_shape=(
            jax.ShapeDtypeStruct((N,), jnp.float32),
            jax.ShapeDtypeStruct((N,), jnp.float32),
            jax.ShapeDtypeStruct((N,), jnp.float32),
            jax.ShapeDtypeStruct((E // 8, 8), jnp.float32),
        ),
    )(us, vs, es2, ku, kv, ke, watt)


def _scores_body(e0_hbm, e1_hbm, e2_hbm, au_hbm, av1_hbm, av2_hbm, ae_hbm,
                 s_hbm, pcnt_hbm, psum_hbm,
                 e0v, e1v, e2v, aev, auv, av1v, av2v, sv, onesv, zbuf,
                 sh_cnt, sh_sum):
    cid = lax.axis_index("c")
    sid = lax.axis_index("s")
    wid = cid * NS + sid
    base = wid * CHUNK

    pltpu.sync_copy(e0_hbm.at[pl.ds(base, CHUNK)], e0v.at[pl.ds(0, CHUNK)])
    pltpu.sync_copy(e1_hbm.at[pl.ds(base, CHUNK)], e1v.at[pl.ds(0, CHUNK)])
    pltpu.sync_copy(e2_hbm.at[pl.ds(base, CHUNK)], e2v.at[pl.ds(0, CHUNK)])
    pltpu.sync_copy(ae_hbm.at[pl.ds(base, CHUNK)], aev.at[pl.ds(0, CHUNK)])
    pltpu.sync_copy(au_hbm, auv.at[pl.ds(0, N)])
    pltpu.sync_copy(av1_hbm, av1v.at[pl.ds(0, N)])
    pltpu.sync_copy(av2_hbm, av2v.at[pl.ds(0, N)])

    zero16 = jnp.zeros((L,), jnp.float32)
    one16 = jnp.ones((L,), jnp.float32)
    lane = lax.iota(jnp.int32, L)
    mtail = lane < TAIL

    def _fill(i, _):
        zbuf[pl.ds(i * L, L)] = zero16
        return 0
    lax.fori_loop(0, 40, _fill, 0)

    def _fill_ones(i, _):
        onesv[pl.ds(i * L, L)] = one16
        return 0
    lax.fori_loop(0, NFULL, _fill_ones, 0)
    onesv[pl.ds(NFULL * L, L)] = jnp.where(mtail, 1.0, 0.0)

    pltpu.sync_copy(zbuf, sh_cnt.at[pl.ds(sid * 640, 640)])
    pltpu.sync_copy(zbuf, sh_sum.at[pl.ds(sid * 640, 640)])

    def _score_vec(i0, i1, i2, ea):
        a = (plsc.load_gather(auv, [i0]) + plsc.load_gather(av1v, [i1])
             + plsc.load_gather(av2v, [i2]) + ea)
        a = jnp.where(a >= 0.0, a, 0.2 * a)
        return jnp.exp(jnp.clip(a, -2.0, 2.0))

    def _score(j, _):
        off = j * L
        sv[pl.ds(off, L)] = _score_vec(
            e0v[pl.ds(off, L)], e1v[pl.ds(off, L)], e2v[pl.ds(off, L)],
            aev[pl.ds(off, L)])
        return 0
    lax.fori_loop(0, NFULL, _score, 0)

    off = NFULL * L
    t0 = jnp.where(mtail, e0v[pl.ds(off, L)], 0)
    t1 = jnp.where(mtail, e1v[pl.ds(off, L)], 0)
    t2 = jnp.where(mtail, e2v[pl.ds(off, L)], 0)
    sv[pl.ds(off, L)] = _score_vec(t0, t1, t2, aev[pl.ds(off, L)])
    e0v[pl.ds(off, L)] = jnp.where(mtail, t0, N)

    pltpu.sync_copy(sv.at[pl.ds(0, CHUNK)], s_hbm.at[pl.ds(base, CHUNK)])

    plsc.subcore_barrier()
    pltpu.sync_copy(onesv, sh_cnt.at[e0v], add=True)
    pltpu.sync_copy(sv, sh_sum.at[e0v], add=True)
    plsc.subcore_barrier()

    pltpu.sync_copy(sh_cnt.at[pl.ds(sid * 640, 640)],
                    pcnt_hbm.at[pl.ds(cid * N_PAD + sid * 640, 640)])
    pltpu.sync_copy(sh_sum.at[pl.ds(sid * 640, 640)],
                    psum_hbm.at[pl.ds(cid * N_PAD + sid * 640, 640)])


def _scores(e0, e1, e2, au, av1, av2, ae):
    f = pl.kernel(
        _scores_body,
        out_type=(
            jax.ShapeDtypeStruct((E,), jnp.float32),
            jax.ShapeDtypeStruct((NC * N_PAD,), jnp.float32),
            jax.ShapeDtypeStruct((NC * N_PAD,), jnp.float32),
        ),
        mesh=_mesh,
        compiler_params=_sc_params,
        scratch_types=[
            pltpu.VMEM((CAP,), jnp.int32),
            pltpu.VMEM((CAP,), jnp.int32),
            pltpu.VMEM((CAP,), jnp.int32),
            pltpu.VMEM((CAP,), jnp.float32),
            pltpu.VMEM((N_PAD,), jnp.float32),
            pltpu.VMEM((N_PAD,), jnp.float32),
            pltpu.VMEM((N_PAD,), jnp.float32),
            pltpu.VMEM((CAP,), jnp.float32),
            pltpu.VMEM((CAP,), jnp.float32),
            pltpu.VMEM((640,), jnp.float32),
            pltpu.VMEM_SHARED((N_PAD,), jnp.float32),
            pltpu.VMEM_SHARED((N_PAD,), jnp.float32),
        ],
    )
    return f(e0, e1, e2, au, av1, av2, ae)


def _scatter_body(pcnt_hbm, psum_hbm, s_hbm, e0_hbm, es_hbm,
                  pacc_hbm,
                  ccv, ssumv, sv, e0v, rows, psv, zbuf,
                  sh_acc):
    cid = lax.axis_index("c")
    sid = lax.axis_index("s")
    wid = cid * NS + sid
    base = wid * CHUNK
    lane = lax.iota(jnp.int32, L)
    mtail = lane < TAIL

    pltpu.sync_copy(pcnt_hbm.at[pl.ds(0, NG)], ccv)
    pltpu.sync_copy(pcnt_hbm.at[pl.ds(N_PAD, NG)], ssumv)

    def _cc(j, carry):
        c = ccv[pl.ds(j * L, L)] + ssumv[pl.ds(j * L, L)]
        ccv[pl.ds(j * L, L)] = plsc.cumsum(c) + carry
        return carry + jnp.sum(c)
    lax.fori_loop(0, NG // L, _cc, jnp.float32(0.0))

    pltpu.sync_copy(psum_hbm.at[pl.ds(0, NG)], ssumv)
    pltpu.sync_copy(psum_hbm.at[pl.ds(N_PAD, NG)], psv)
    pltpu.sync_copy(s_hbm.at[pl.ds(base, CHUNK)], sv.at[pl.ds(0, CHUNK)])
    pltpu.sync_copy(e0_hbm.at[pl.ds(base, CHUNK)], e0v.at[pl.ds(0, CHUNK)])
    pltpu.sync_copy(es_hbm.at[pl.ds(base, CHUNK), :], rows.at[pl.ds(0, CHUNK), :])

    zero16 = jnp.zeros((L,), jnp.float32)

    def _fill(i, _):
        zbuf[i, :] = zero16
        return 0
    lax.fori_loop(0, 32, _fill, 0)

    for t in range(20):
        pltpu.sync_copy(zbuf, sh_acc.at[pl.ds(sid * 640 + t * 32, 32), :])

    def _search(j, _):
        off = j * L
        pos = (base + off + lane).astype(jnp.float32)
        lo = jnp.zeros((L,), jnp.int32)
        step = 8192
        while step >= 1:
            cand = lo + step
            idx = jnp.minimum(cand, N) - 1
            v = plsc.load_gather(ccv, [idx])
            take = (cand <= N) & (v <= pos)
            lo = jnp.where(take, cand, lo)
            step //= 2
        rep = plsc.load_gather(ssumv, [lo]) + plsc.load_gather(psv, [lo])
        sv[pl.ds(off, L)] = sv[pl.ds(off, L)] / rep
        return 0
    lax.fori_loop(0, NFULL + 1, _search, 0)

    off = NFULL * L
    e0v[pl.ds(off, L)] = jnp.where(mtail, e0v[pl.ds(off, L)], N)

    def _scale(j, _):
        wv = sv[pl.ds(j * L, L)]
        for r2 in range(L):
            r = j * L + r2
            rows[r, :] = rows[r, :] * jnp.full((L,), wv[r2])
        return 0
    lax.fori_loop(0, NFULL, _scale, 0)
    wtail = sv[pl.ds(off, L)]
    for r2 in range(TAIL):
        rows[off + r2, :] = rows[off + r2, :] * jnp.full((L,), wtail[r2])

    plsc.subcore_barrier()
    pltpu.sync_copy(rows, sh_acc.at[e0v], add=True)
    plsc.subcore_barrier()

    pltpu.sync_copy(sh_acc.at[pl.ds(sid * 640, 640), :],
                    pacc_hbm.at[pl.ds(cid * N_PAD + sid * 640, 640), :])


def _scatter(pcnt, psum, s, e0, es):
    f = pl.kernel(
        _scatter_body,
        out_type=jax.ShapeDtypeStruct((NC * N_PAD, DE), jnp.float32),
        mesh=_mesh,
        compiler_params=_sc_params,
        scratch_types=[
            pltpu.VMEM((NG,), jnp.float32),
            pltpu.VMEM((NG,), jnp.float32),
            pltpu.VMEM((CAP,), jnp.float32),
            pltpu.VMEM((CAP,), jnp.int32),
            pltpu.VMEM((CAP, DE), jnp.float32),
            pltpu.VMEM((NG,), jnp.float32),
            pltpu.VMEM((32, DE), jnp.float32),
            pltpu.VMEM_SHARED((N_PAD, DE), jnp.float32),
        ],
    )
    return f(pcnt, psum, s, e0, es)


def _out_body(pacc_ref, ke_ref, out_ref):
    acc = pacc_ref[0, :, :] + pacc_ref[1, :, :]
    res = jnp.dot(acc, ke_ref[...], precision=_HI)
    out_ref[...] = res[0:N, :]


def _out_mm(pacc, ke):
    return pl.pallas_call(
        _out_body,
        out_shape=jax.ShapeDtypeStruct((N, U), jnp.float32),
    )(pacc, ke)


def kernel(user_states, video_states, edge_states, hyper_edges,
           kernel_user, kernel_video, kernel_edge, kernel_user_attention):
    e0 = hyper_edges[:, 0]
    e1 = hyper_edges[:, 1]
    e2 = hyper_edges[:, 2]
    lin = lax.optimization_barrier(edge_states.reshape(E * DE))
    es2 = lin.reshape(E // 8, DN)
    esd = lin.reshape(E, DE)

    au, av1, av2, ae2 = _proj(user_states, video_states, es2, kernel_user,
                              kernel_video, kernel_edge,
                              kernel_user_attention)
    s, pcnt, psum = _scores(e0, e1, e2, au, av1, av2, ae2.reshape(E))
    pacc = _scatter(pcnt, psum, s, e0, esd)
    return _out_mm(pacc.reshape(NC, N_PAD, DE), kernel_edge)


# speedup vs baseline: 1.3833x; 1.0427x over previous
import jax
import jax.numpy as jnp
from jax import lax
from jax.experimental import pallas as pl
from jax.experimental.pallas import tpu as pltpu
from jax.experimental.pallas import tpu_sc as plsc

N = 10000
E = 160000
DN = 128
DE = 16
U = 64

NC, NS, L = 2, 16, 16
NW = NC * NS
N_PAD = 10240
CHUNK = E // NW
CAP = 5008
NG = 10016
NFULL = CHUNK // L
TAIL = CHUNK - NFULL * L
EB = 16000

_mesh = plsc.VectorSubcoreMesh(core_axis_name="c", subcore_axis_name="s")
_HI = lax.Precision.HIGHEST
_sc_params = pltpu.CompilerParams(needs_layout_passes=False,
                                  use_tc_tiling_on_sc=False)


def _proj_body(us_ref, vs_ref, es2_ref, ku_ref, kv_ref, ke_ref, watt_ref,
               au_ref, av1_ref, av2_ref, ae_ref):
    i = pl.program_id(0)
    watt = watt_ref[...]
    we = jnp.dot(ke_ref[...], watt[192:256], precision=_HI)[:, 0]
    wrep = jnp.concatenate([we] * 8)
    r = lax.broadcasted_iota(jnp.int32, (DN, 8), 0)
    c = lax.broadcasted_iota(jnp.int32, (DN, 8), 1)
    wmat = jnp.where((r // DE) == c, wrep[:, None], 0.0)
    ae_ref[...] = jnp.dot(es2_ref[...], wmat, precision=_HI)

    @pl.when(i == 0)
    def _node_scalars():
        wu = jnp.dot(ku_ref[...], watt[0:64], precision=_HI)[:, 0]
        wv1 = jnp.dot(kv_ref[...], watt[64:128], precision=_HI)[:, 0]
        wv2 = jnp.dot(kv_ref[...], watt[128:192], precision=_HI)[:, 0]
        us = us_ref[...]
        vs = vs_ref[...]
        au_ref[...] = jnp.sum(us * wu[None, :], axis=1)
        av1_ref[...] = jnp.sum(vs * wv1[None, :], axis=1)
        av2_ref[...] = jnp.sum(vs * wv2[None, :], axis=1)


def _proj(us, vs, es2, ku, kv, ke, watt):
    g = 10
    eb = E // 8 // g
    return pl.pallas_call(
        _proj_body,
        grid=(g,),
        in_specs=[
            pl.BlockSpec((N, DN), lambda i: (0, 0)),
            pl.BlockSpec((N, DN), lambda i: (0, 0)),
            pl.BlockSpec((eb, DN), lambda i: (i, 0)),
            pl.BlockSpec((DN, U), lambda i: (0, 0)),
            pl.BlockSpec((DN, U), lambda i: (0, 0)),
            pl.BlockSpec((DE, U), lambda i: (0, 0)),
            pl.BlockSpec((4 * U, 1), lambda i: (0, 0)),
        ],
        out_specs=(
            pl.BlockSpec((N,), lambda i: (0,)),
            pl.BlockSpec((N,), lambda i: (0,)),
            pl.BlockSpec((N,), lambda i: (0,)),
            pl.BlockSpec((eb, 8), lambda i: (i, 0)),
        ),
        out_shape=(
            jax.ShapeDtypeStruct((N,), jnp.float32),
            jax.ShapeDtypeStruct((N,), jnp.float32),
            jax.ShapeDtypeStruct((N,), jnp.float32),
            jax.ShapeDtypeStruct((E // 8, 8), jnp.float32),
        ),
    )(us, vs, es2, ku, kv, ke, watt)


def _scores_body(e0_hbm, e1_hbm, e2_hbm, au_hbm, av1_hbm, av2_hbm, ae_hbm,
                 s_hbm, pcnt_hbm, psum_hbm,
                 e0v, e1v, e2v, aev, auv, av1v, av2v, sv, onesv, zbuf,
                 sh_cnt, sh_sum, sems):
    cid = lax.axis_index("c")
    sid = lax.axis_index("s")
    wid = cid * NS + sid
    base = wid * CHUNK

    cps = [
        pltpu.async_copy(e0_hbm.at[pl.ds(base, CHUNK)],
                         e0v.at[pl.ds(0, CHUNK)], sems.at[0]),
        pltpu.async_copy(e1_hbm.at[pl.ds(base, CHUNK)],
                         e1v.at[pl.ds(0, CHUNK)], sems.at[1]),
        pltpu.async_copy(e2_hbm.at[pl.ds(base, CHUNK)],
                         e2v.at[pl.ds(0, CHUNK)], sems.at[2]),
        pltpu.async_copy(ae_hbm.at[pl.ds(base, CHUNK)],
                         aev.at[pl.ds(0, CHUNK)], sems.at[3]),
        pltpu.async_copy(au_hbm, auv.at[pl.ds(0, N)], sems.at[4]),
        pltpu.async_copy(av1_hbm, av1v.at[pl.ds(0, N)], sems.at[5]),
        pltpu.async_copy(av2_hbm, av2v.at[pl.ds(0, N)], sems.at[6]),
    ]

    zero16 = jnp.zeros((L,), jnp.float32)
    one16 = jnp.ones((L,), jnp.float32)
    lane = lax.iota(jnp.int32, L)
    mtail = lane < TAIL

    def _fill(i, _):
        zbuf[pl.ds(i * L, L)] = zero16
        return 0
    lax.fori_loop(0, 40, _fill, 0)

    def _fill_ones(i, _):
        onesv[pl.ds(i * L, L)] = one16
        return 0
    lax.fori_loop(0, NFULL, _fill_ones, 0)
    onesv[pl.ds(NFULL * L, L)] = jnp.where(mtail, 1.0, 0.0)

    pltpu.sync_copy(zbuf, sh_cnt.at[pl.ds(sid * 640, 640)])
    pltpu.sync_copy(zbuf, sh_sum.at[pl.ds(sid * 640, 640)])
    for cp in cps:
        cp.wait()

    def _score_vec(i0, i1, i2, ea):
        a = (plsc.load_gather(auv, [i0]) + plsc.load_gather(av1v, [i1])
             + plsc.load_gather(av2v, [i2]) + ea)
        a = jnp.where(a >= 0.0, a, 0.2 * a)
        return jnp.exp(jnp.clip(a, -2.0, 2.0))

    def _score(j, _):
        off = j * L
        sv[pl.ds(off, L)] = _score_vec(
            e0v[pl.ds(off, L)], e1v[pl.ds(off, L)], e2v[pl.ds(off, L)],
            aev[pl.ds(off, L)])
        return 0
    lax.fori_loop(0, NFULL, _score, 0)

    off = NFULL * L
    t0 = jnp.where(mtail, e0v[pl.ds(off, L)], 0)
    t1 = jnp.where(mtail, e1v[pl.ds(off, L)], 0)
    t2 = jnp.where(mtail, e2v[pl.ds(off, L)], 0)
    sv[pl.ds(off, L)] = _score_vec(t0, t1, t2, aev[pl.ds(off, L)])
    e0v[pl.ds(off, L)] = jnp.where(mtail, t0, N)

    s_cp = pltpu.async_copy(sv.at[pl.ds(0, CHUNK)],
                            s_hbm.at[pl.ds(base, CHUNK)], sems.at[7])

    plsc.subcore_barrier()
    pltpu.sync_copy(onesv, sh_cnt.at[e0v], add=True)
    pltpu.sync_copy(sv, sh_sum.at[e0v], add=True)
    plsc.subcore_barrier()

    pltpu.sync_copy(sh_cnt.at[pl.ds(sid * 640, 640)],
                    pcnt_hbm.at[pl.ds(cid * N_PAD + sid * 640, 640)])
    pltpu.sync_copy(sh_sum.at[pl.ds(sid * 640, 640)],
                    psum_hbm.at[pl.ds(cid * N_PAD + sid * 640, 640)])
    s_cp.wait()


def _scores(e0, e1, e2, au, av1, av2, ae):
    f = pl.kernel(
        _scores_body,
        out_type=(
            jax.ShapeDtypeStruct((E,), jnp.float32),
            jax.ShapeDtypeStruct((NC * N_PAD,), jnp.float32),
            jax.ShapeDtypeStruct((NC * N_PAD,), jnp.float32),
        ),
        mesh=_mesh,
        compiler_params=_sc_params,
        scratch_types=[
            pltpu.VMEM((CAP,), jnp.int32),
            pltpu.VMEM((CAP,), jnp.int32),
            pltpu.VMEM((CAP,), jnp.int32),
            pltpu.VMEM((CAP,), jnp.float32),
            pltpu.VMEM((N_PAD,), jnp.float32),
            pltpu.VMEM((N_PAD,), jnp.float32),
            pltpu.VMEM((N_PAD,), jnp.float32),
            pltpu.VMEM((CAP,), jnp.float32),
            pltpu.VMEM((CAP,), jnp.float32),
            pltpu.VMEM((640,), jnp.float32),
            pltpu.VMEM_SHARED((N_PAD,), jnp.float32),
            pltpu.VMEM_SHARED((N_PAD,), jnp.float32),
            pltpu.SemaphoreType.DMA((8,)),
        ],
    )
    return f(e0, e1, e2, au, av1, av2, ae)


def _scatter_body(pcnt_hbm, psum_hbm, s_hbm, e0_hbm, es_hbm,
                  pacc_hbm,
                  ccv, ssumv, sv, e0v, rows, psv, zbuf,
                  sh_acc, sems):
    cid = lax.axis_index("c")
    sid = lax.axis_index("s")
    wid = cid * NS + sid
    base = wid * CHUNK
    lane = lax.iota(jnp.int32, L)
    mtail = lane < TAIL

    cp_c0 = pltpu.async_copy(pcnt_hbm.at[pl.ds(0, NG)], ccv, sems.at[0])
    cp_c1 = pltpu.async_copy(pcnt_hbm.at[pl.ds(N_PAD, NG)], ssumv, sems.at[1])
    cp_p0 = pltpu.async_copy(psum_hbm.at[pl.ds(0, NG)], psv, sems.at[2])
    cp_s = pltpu.async_copy(s_hbm.at[pl.ds(base, CHUNK)],
                            sv.at[pl.ds(0, CHUNK)], sems.at[3])
    cp_e0 = pltpu.async_copy(e0_hbm.at[pl.ds(base, CHUNK)],
                             e0v.at[pl.ds(0, CHUNK)], sems.at[4])
    cp_rows = pltpu.async_copy(es_hbm.at[pl.ds(base, CHUNK), :],
                               rows.at[pl.ds(0, CHUNK), :], sems.at[5])
    cp_c0.wait()
    cp_c1.wait()

    def _cc(j, carry):
        c = ccv[pl.ds(j * L, L)] + ssumv[pl.ds(j * L, L)]
        ccv[pl.ds(j * L, L)] = plsc.cumsum(c) + carry
        return carry + jnp.sum(c)
    lax.fori_loop(0, NG // L, _cc, jnp.float32(0.0))

    pltpu.sync_copy(psum_hbm.at[pl.ds(N_PAD, NG)], ssumv)

    zero16 = jnp.zeros((L,), jnp.float32)

    def _fill(i, _):
        zbuf[i, :] = zero16
        return 0
    lax.fori_loop(0, 32, _fill, 0)

    for t in range(20):
        pltpu.sync_copy(zbuf, sh_acc.at[pl.ds(sid * 640 + t * 32, 32), :])

    cp_p0.wait()
    cp_s.wait()
    cp_e0.wait()
    cp_rows.wait()

    def _search(j, _):
        off = j * L
        pos = (base + off + lane).astype(jnp.float32)
        lo = jnp.zeros((L,), jnp.int32)
        step = 8192
        while step >= 1:
            cand = lo + step
            idx = jnp.minimum(cand, N) - 1
            v = plsc.load_gather(ccv, [idx])
            take = (cand <= N) & (v <= pos)
            lo = jnp.where(take, cand, lo)
            step //= 2
        rep = plsc.load_gather(psv, [lo]) + plsc.load_gather(ssumv, [lo])
        sv[pl.ds(off, L)] = sv[pl.ds(off, L)] / rep
        return 0
    lax.fori_loop(0, NFULL + 1, _search, 0)

    off = NFULL * L
    e0v[pl.ds(off, L)] = jnp.where(mtail, e0v[pl.ds(off, L)], N)

    def _scale(j, _):
        wv = sv[pl.ds(j * L, L)]
        for r2 in range(L):
            r = j * L + r2
            rows[r, :] = rows[r, :] * jnp.full((L,), wv[r2])
        return 0
    lax.fori_loop(0, NFULL, _scale, 0)
    wtail = sv[pl.ds(off, L)]
    for r2 in range(TAIL):
        rows[off + r2, :] = rows[off + r2, :] * jnp.full((L,), wtail[r2])

    plsc.subcore_barrier()
    pltpu.sync_copy(rows, sh_acc.at[e0v], add=True)
    plsc.subcore_barrier()

    pltpu.sync_copy(sh_acc.at[pl.ds(sid * 640, 640), :],
                    pacc_hbm.at[pl.ds(cid * N_PAD + sid * 640, 640), :])


def _scatter(pcnt, psum, s, e0, es):
    f = pl.kernel(
        _scatter_body,
        out_type=jax.ShapeDtypeStruct((NC * N_PAD, DE), jnp.float32),
        mesh=_mesh,
        compiler_params=_sc_params,
        scratch_types=[
            pltpu.VMEM((NG,), jnp.float32),
            pltpu.VMEM((NG,), jnp.float32),
            pltpu.VMEM((CAP,), jnp.float32),
            pltpu.VMEM((CAP,), jnp.int32),
            pltpu.VMEM((CAP, DE), jnp.float32),
            pltpu.VMEM((NG,), jnp.float32),
            pltpu.VMEM((32, DE), jnp.float32),
            pltpu.VMEM_SHARED((N_PAD, DE), jnp.float32),
            pltpu.SemaphoreType.DMA((6,)),
        ],
    )
    return f(pcnt, psum, s, e0, es)


def _out_body(pacc_ref, ke_ref, out_ref):
    acc = pacc_ref[0, :, :] + pacc_ref[1, :, :]
    res = jnp.dot(acc, ke_ref[...], precision=_HI)
    out_ref[...] = res[0:N, :]


def _out_mm(pacc, ke):
    return pl.pallas_call(
        _out_body,
        out_shape=jax.ShapeDtypeStruct((N, U), jnp.float32),
    )(pacc, ke)


def kernel(user_states, video_states, edge_states, hyper_edges,
           kernel_user, kernel_video, kernel_edge, kernel_user_attention):
    e0 = hyper_edges[:, 0]
    e1 = hyper_edges[:, 1]
    e2 = hyper_edges[:, 2]
    lin = lax.optimization_barrier(edge_states.reshape(E * DE))
    es2 = lin.reshape(E // 8, DN)
    esd = lin.reshape(E, DE)

    au, av1, av2, ae2 = _proj(user_states, video_states, es2, kernel_user,
                              kernel_video, kernel_edge,
                              kernel_user_attention)
    s, pcnt, psum = _scores(e0, e1, e2, au, av1, av2, ae2.reshape(E))
    pacc = _scatter(pcnt, psum, s, e0, esd)
    return _out_mm(pacc.reshape(NC, N_PAD, DE), kernel_edge)


# speedup vs baseline: 1.3891x; 1.0042x over previous
import jax
import jax.numpy as jnp
from jax import lax
from jax.experimental import pallas as pl
from jax.experimental.pallas import tpu as pltpu
from jax.experimental.pallas import tpu_sc as plsc

N = 10000
E = 160000
DN = 128
DE = 16
U = 64

NC, NS, L = 2, 16, 16
NW = NC * NS
N_PAD = 10240
CHUNK = E // NW
CAP = 5008
NG = 10016
NFULL = CHUNK // L
TAIL = CHUNK - NFULL * L
H0 = 2512
H1 = CHUNK - H0
H1CAP = 2496
EB = 16000

_mesh = plsc.VectorSubcoreMesh(core_axis_name="c", subcore_axis_name="s")
_HI = lax.Precision.HIGHEST
_sc_params = pltpu.CompilerParams(needs_layout_passes=False,
                                  use_tc_tiling_on_sc=False)


def _proj_body(us_ref, vs_ref, es2_ref, ku_ref, kv_ref, ke_ref, watt_ref,
               au_ref, av1_ref, av2_ref, ae_ref):
    i = pl.program_id(0)
    watt = watt_ref[...]
    we = jnp.dot(ke_ref[...], watt[192:256], precision=_HI)[:, 0]
    wrep = jnp.concatenate([we] * 8)
    r = lax.broadcasted_iota(jnp.int32, (DN, 8), 0)
    c = lax.broadcasted_iota(jnp.int32, (DN, 8), 1)
    wmat = jnp.where((r // DE) == c, wrep[:, None], 0.0)
    ae_ref[...] = jnp.dot(es2_ref[...], wmat, precision=_HI)

    @pl.when(i == 0)
    def _node_scalars():
        wu = jnp.dot(ku_ref[...], watt[0:64], precision=_HI)[:, 0]
        wv1 = jnp.dot(kv_ref[...], watt[64:128], precision=_HI)[:, 0]
        wv2 = jnp.dot(kv_ref[...], watt[128:192], precision=_HI)[:, 0]
        us = us_ref[...]
        vs = vs_ref[...]
        au_ref[...] = jnp.sum(us * wu[None, :], axis=1)
        av1_ref[...] = jnp.sum(vs * wv1[None, :], axis=1)
        av2_ref[...] = jnp.sum(vs * wv2[None, :], axis=1)


def _proj(us, vs, es2, ku, kv, ke, watt):
    g = 10
    eb = E // 8 // g
    return pl.pallas_call(
        _proj_body,
        grid=(g,),
        in_specs=[
            pl.BlockSpec((N, DN), lambda i: (0, 0)),
            pl.BlockSpec((N, DN), lambda i: (0, 0)),
            pl.BlockSpec((eb, DN), lambda i: (i, 0)),
            pl.BlockSpec((DN, U), lambda i: (0, 0)),
            pl.BlockSpec((DN, U), lambda i: (0, 0)),
            pl.BlockSpec((DE, U), lambda i: (0, 0)),
            pl.BlockSpec((4 * U, 1), lambda i: (0, 0)),
        ],
        out_specs=(
            pl.BlockSpec((N,), lambda i: (0,)),
            pl.BlockSpec((N,), lambda i: (0,)),
            pl.BlockSpec((N,), lambda i: (0,)),
            pl.BlockSpec((eb, 8), lambda i: (i, 0)),
        ),
        out_shape=(
            jax.ShapeDtypeStruct((N,), jnp.float32),
            jax.ShapeDtypeStruct((N,), jnp.float32),
            jax.ShapeDtypeStruct((N,), jnp.float32),
            jax.ShapeDtypeStruct((E // 8, 8), jnp.float32),
        ),
    )(us, vs, es2, ku, kv, ke, watt)


def _scores_body(e0_hbm, e1_hbm, e2_hbm, au_hbm, av1_hbm, av2_hbm, ae_hbm,
                 s_hbm, pcnt_hbm, psum_hbm,
                 e0v, e1v, e2v, aev, auv, av1v, av2v, sv, onesv, zbuf,
                 sh_cnt, sh_sum, sems):
    cid = lax.axis_index("c")
    sid = lax.axis_index("s")
    wid = cid * NS + sid
    base = wid * CHUNK

    cps = [
        pltpu.async_copy(e0_hbm.at[pl.ds(base, CHUNK)],
                         e0v.at[pl.ds(0, CHUNK)], sems.at[0]),
        pltpu.async_copy(e1_hbm.at[pl.ds(base, CHUNK)],
                         e1v.at[pl.ds(0, CHUNK)], sems.at[1]),
        pltpu.async_copy(e2_hbm.at[pl.ds(base, CHUNK)],
                         e2v.at[pl.ds(0, CHUNK)], sems.at[2]),
        pltpu.async_copy(ae_hbm.at[pl.ds(base, CHUNK)],
                         aev.at[pl.ds(0, CHUNK)], sems.at[3]),
        pltpu.async_copy(au_hbm, auv.at[pl.ds(0, N)], sems.at[4]),
        pltpu.async_copy(av1_hbm, av1v.at[pl.ds(0, N)], sems.at[5]),
        pltpu.async_copy(av2_hbm, av2v.at[pl.ds(0, N)], sems.at[6]),
    ]

    zero16 = jnp.zeros((L,), jnp.float32)
    one16 = jnp.ones((L,), jnp.float32)
    lane = lax.iota(jnp.int32, L)
    mtail = lane < TAIL

    def _fill(i, _):
        zbuf[pl.ds(i * L, L)] = zero16
        return 0
    lax.fori_loop(0, 40, _fill, 0)

    def _fill_ones(i, _):
        onesv[pl.ds(i * L, L)] = one16
        return 0
    lax.fori_loop(0, NFULL, _fill_ones, 0)
    onesv[pl.ds(NFULL * L, L)] = jnp.where(mtail, 1.0, 0.0)

    pltpu.sync_copy(zbuf, sh_cnt.at[pl.ds(sid * 640, 640)])
    pltpu.sync_copy(zbuf, sh_sum.at[pl.ds(sid * 640, 640)])
    for cp in cps:
        cp.wait()

    def _score_vec(i0, i1, i2, ea):
        a = (plsc.load_gather(auv, [i0]) + plsc.load_gather(av1v, [i1])
             + plsc.load_gather(av2v, [i2]) + ea)
        a = jnp.where(a >= 0.0, a, 0.2 * a)
        return jnp.exp(jnp.clip(a, -2.0, 2.0))

    def _score(j, _):
        off = j * L
        sv[pl.ds(off, L)] = _score_vec(
            e0v[pl.ds(off, L)], e1v[pl.ds(off, L)], e2v[pl.ds(off, L)],
            aev[pl.ds(off, L)])
        return 0
    lax.fori_loop(0, NFULL, _score, 0)

    off = NFULL * L
    t0 = jnp.where(mtail, e0v[pl.ds(off, L)], 0)
    t1 = jnp.where(mtail, e1v[pl.ds(off, L)], 0)
    t2 = jnp.where(mtail, e2v[pl.ds(off, L)], 0)
    sv[pl.ds(off, L)] = _score_vec(t0, t1, t2, aev[pl.ds(off, L)])
    e0v[pl.ds(off, L)] = jnp.where(mtail, t0, N)

    s_cp = pltpu.async_copy(sv.at[pl.ds(0, CHUNK)],
                            s_hbm.at[pl.ds(base, CHUNK)], sems.at[7])

    plsc.subcore_barrier()
    pltpu.sync_copy(onesv, sh_cnt.at[e0v], add=True)
    pltpu.sync_copy(sv, sh_sum.at[e0v], add=True)
    plsc.subcore_barrier()

    pltpu.sync_copy(sh_cnt.at[pl.ds(sid * 640, 640)],
                    pcnt_hbm.at[pl.ds(cid * N_PAD + sid * 640, 640)])
    pltpu.sync_copy(sh_sum.at[pl.ds(sid * 640, 640)],
                    psum_hbm.at[pl.ds(cid * N_PAD + sid * 640, 640)])
    s_cp.wait()


def _scores(e0, e1, e2, au, av1, av2, ae):
    f = pl.kernel(
        _scores_body,
        out_type=(
            jax.ShapeDtypeStruct((E,), jnp.float32),
            jax.ShapeDtypeStruct((NC * N_PAD,), jnp.float32),
            jax.ShapeDtypeStruct((NC * N_PAD,), jnp.float32),
        ),
        mesh=_mesh,
        compiler_params=_sc_params,
        scratch_types=[
            pltpu.VMEM((CAP,), jnp.int32),
            pltpu.VMEM((CAP,), jnp.int32),
            pltpu.VMEM((CAP,), jnp.int32),
            pltpu.VMEM((CAP,), jnp.float32),
            pltpu.VMEM((N_PAD,), jnp.float32),
            pltpu.VMEM((N_PAD,), jnp.float32),
            pltpu.VMEM((N_PAD,), jnp.float32),
            pltpu.VMEM((CAP,), jnp.float32),
            pltpu.VMEM((CAP,), jnp.float32),
            pltpu.VMEM((640,), jnp.float32),
            pltpu.VMEM_SHARED((N_PAD,), jnp.float32),
            pltpu.VMEM_SHARED((N_PAD,), jnp.float32),
            pltpu.SemaphoreType.DMA((8,)),
        ],
    )
    return f(e0, e1, e2, au, av1, av2, ae)


def _scatter_body(pcnt_hbm, psum_hbm, s_hbm, e0_hbm, es_hbm,
                  pacc_hbm,
                  ccv, ssumv, sv, e0a, e0b, rows, psv, zbuf,
                  sh_acc, sems):
    cid = lax.axis_index("c")
    sid = lax.axis_index("s")
    wid = cid * NS + sid
    base = wid * CHUNK
    lane = lax.iota(jnp.int32, L)
    mtail = lane < TAIL

    cp_c0 = pltpu.async_copy(pcnt_hbm.at[pl.ds(0, NG)], ccv, sems.at[0])
    cp_c1 = pltpu.async_copy(pcnt_hbm.at[pl.ds(N_PAD, NG)], ssumv, sems.at[1])
    cp_p0 = pltpu.async_copy(psum_hbm.at[pl.ds(0, NG)], psv, sems.at[2])
    cp_s = pltpu.async_copy(s_hbm.at[pl.ds(base, CHUNK)],
                            sv.at[pl.ds(0, CHUNK)], sems.at[3])
    cp_e0a = pltpu.async_copy(e0_hbm.at[pl.ds(base, H0)], e0a, sems.at[4])
    cp_e0b = pltpu.async_copy(e0_hbm.at[pl.ds(base + H0, H1)],
                              e0b.at[pl.ds(0, H1)], sems.at[5])
    cp_rows = pltpu.async_copy(es_hbm.at[pl.ds(base, CHUNK), :],
                               rows.at[pl.ds(0, CHUNK), :], sems.at[6])
    cp_c0.wait()
    cp_c1.wait()

    def _cc(j, carry):
        c = ccv[pl.ds(j * L, L)] + ssumv[pl.ds(j * L, L)]
        ccv[pl.ds(j * L, L)] = plsc.cumsum(c) + carry
        return carry + jnp.sum(c)
    lax.fori_loop(0, NG // L, _cc, jnp.float32(0.0))

    pltpu.sync_copy(psum_hbm.at[pl.ds(N_PAD, NG)], ssumv)

    zero16 = jnp.zeros((L,), jnp.float32)

    def _fill(i, _):
        zbuf[i, :] = zero16
        return 0
    lax.fori_loop(0, 32, _fill, 0)

    for t in range(20):
        pltpu.sync_copy(zbuf, sh_acc.at[pl.ds(sid * 640 + t * 32, 32), :])
    plsc.subcore_barrier()

    cp_p0.wait()
    cp_s.wait()
    cp_rows.wait()

    def _search(j, _):
        off = j * L
        pos = (base + off + lane).astype(jnp.float32)
        lo = jnp.zeros((L,), jnp.int32)
        step = 8192
        while step >= 1:
            cand = lo + step
            idx = jnp.minimum(cand, N) - 1
            v = plsc.load_gather(ccv, [idx])
            take = (cand <= N) & (v <= pos)
            lo = jnp.where(take, cand, lo)
            step //= 2
        rep = plsc.load_gather(psv, [lo]) + plsc.load_gather(ssumv, [lo])
        sv[pl.ds(off, L)] = sv[pl.ds(off, L)] / rep
        return 0
    lax.fori_loop(0, NFULL + 1, _search, 0)

    cp_e0a.wait()
    cp_e0b.wait()
    tb = NFULL * L - H0
    e0b[pl.ds(tb, L)] = jnp.where(mtail, e0b[pl.ds(tb, L)], N)

    def _scale0(j, _):
        wv = sv[pl.ds(j * L, L)]
        for r2 in range(L):
            r = j * L + r2
            rows[r, :] = rows[r, :] * jnp.full((L,), wv[r2])
        return 0
    lax.fori_loop(0, H0 // L, _scale0, 0)
    st1 = pltpu.async_copy(rows.at[pl.ds(0, H0), :], sh_acc.at[e0a],
                           sems.at[7], add=True)

    def _scale1(j, _):
        off1 = H0 + j * L
        wv = sv[pl.ds(off1, L)]
        for r2 in range(L):
            r = off1 + r2
            rows[r, :] = rows[r, :] * jnp.full((L,), wv[r2])
        return 0
    lax.fori_loop(0, (NFULL * L - H0) // L, _scale1, 0)
    off = NFULL * L
    wtail = sv[pl.ds(off, L)]
    for r2 in range(TAIL):
        rows[off + r2, :] = rows[off + r2, :] * jnp.full((L,), wtail[r2])

    st2 = pltpu.async_copy(rows.at[pl.ds(H0, H1CAP), :], sh_acc.at[e0b],
                           sems.at[2], add=True)
    st1.wait()
    st2.wait()
    plsc.subcore_barrier()

    pltpu.sync_copy(sh_acc.at[pl.ds(sid * 640, 640), :],
                    pacc_hbm.at[pl.ds(cid * N_PAD + sid * 640, 640), :])


def _scatter(pcnt, psum, s, e0, es):
    f = pl.kernel(
        _scatter_body,
        out_type=jax.ShapeDtypeStruct((NC * N_PAD, DE), jnp.float32),
        mesh=_mesh,
        compiler_params=_sc_params,
        scratch_types=[
            pltpu.VMEM((NG,), jnp.float32),
            pltpu.VMEM((NG,), jnp.float32),
            pltpu.VMEM((CAP,), jnp.float32),
            pltpu.VMEM((H0,), jnp.int32),
            pltpu.VMEM((H1CAP,), jnp.int32),
            pltpu.VMEM((CAP, DE), jnp.float32),
            pltpu.VMEM((NG,), jnp.float32),
            pltpu.VMEM((32, DE), jnp.float32),
            pltpu.VMEM_SHARED((N_PAD, DE), jnp.float32),
            pltpu.SemaphoreType.DMA((8,)),
        ],
    )
    return f(pcnt, psum, s, e0, es)


def _out_body(pacc_ref, ke_ref, out_ref):
    acc = pacc_ref[0, :, :] + pacc_ref[1, :, :]
    res = jnp.dot(acc, ke_ref[...], precision=_HI)
    out_ref[...] = res[0:N, :]


def _out_mm(pacc, ke):
    return pl.pallas_call(
        _out_body,
        out_shape=jax.ShapeDtypeStruct((N, U), jnp.float32),
    )(pacc, ke)


def kernel(user_states, video_states, edge_states, hyper_edges,
           kernel_user, kernel_video, kernel_edge, kernel_user_attention):
    e0 = hyper_edges[:, 0]
    e1 = hyper_edges[:, 1]
    e2 = hyper_edges[:, 2]
    lin = lax.optimization_barrier(edge_states.reshape(E * DE))
    es2 = lin.reshape(E // 8, DN)
    esd = lin.reshape(E, DE)

    au, av1, av2, ae2 = _proj(user_states, video_states, es2, kernel_user,
                              kernel_video, kernel_edge,
                              kernel_user_attention)
    s, pcnt, psum = _scores(e0, e1, e2, au, av1, av2, ae2.reshape(E))
    pacc = _scatter(pcnt, psum, s, e0, esd)
    return _out_mm(pacc.reshape(NC, N_PAD, DE), kernel_edge)
